# Initial kernel scaffold; baseline (speedup 1.0000x reference)
import jax
import jax.numpy as jnp
from jax.experimental import pallas as pl


def kernel(x, W1, b1, W2, b2):
    raise NotImplementedError("write your pallas kernel here")


# speedup vs baseline: 4.7542x; 4.7542x over previous
import functools

import jax
import jax.numpy as jnp
from jax import lax
from jax.experimental import pallas as pl
from jax.experimental.pallas import tpu as pltpu
from jax.experimental.pallas import tpu_sc as plsc

N = 10000
D = 128
K = 16
OUT = 128
NPAD = 10240
BQ = 256
NBLK = NPAD // BQ
NW = 32
QPW = NPAD // NW
QB = 8
NBATCH = QPW // QB


def _split_bf16(a):
    hi = a.astype(jnp.bfloat16)
    lo = (a - hi.astype(jnp.float32)).astype(jnp.bfloat16)
    return hi, lo


def _mm3(a, b):
    ah, al = _split_bf16(a)
    bh, bl = _split_bf16(b)
    return (jnp.dot(ah, bh, preferred_element_type=jnp.float32)
            + jnp.dot(ah, bl, preferred_element_type=jnp.float32)
            + jnp.dot(al, bh, preferred_element_type=jnp.float32))


def _prep_body(xp_ref, w1a_ref, w1b_ref, b1_ref, a_ref, b_ref, sq_ref):
    xp = xp_ref[...]
    wd = w1a_ref[...] - w1b_ref[...]
    a_ref[...] = _mm3(xp, wd) + b1_ref[...]
    b_ref[...] = _mm3(xp, w1b_ref[...])
    sq = jnp.sum(xp * xp, axis=1, keepdims=True)
    row = lax.broadcasted_iota(jnp.int32, (NPAD, 1), 0)
    sq_ref[...] = jnp.where(row < N, sq, jnp.float32(1e30))


def _prep_call(xp, w1a, w1b, b1row):
    return pl.pallas_call(
        _prep_body,
        out_shape=(
            jax.ShapeDtypeStruct((NPAD, OUT), jnp.float32),
            jax.ShapeDtypeStruct((NPAD, OUT), jnp.float32),
            jax.ShapeDtypeStruct((NPAD, 1), jnp.float32),
        ),
    )(xp, w1a, w1b, b1row)


def _knn_body(xbf_ref, xbft_ref, sq_ref, idx_ref):
    dot = jnp.dot(xbf_ref[...], xbft_ref[...],
                  preferred_element_type=jnp.float32)
    key = sq_ref[...] - 2.0 * dot
    col = lax.broadcasted_iota(jnp.int32, (BQ, NPAD), 1)
    cols = []
    for _ in range(K):
        m = jnp.min(key, axis=1, keepdims=True)
        j = jnp.min(jnp.where(key == m, col, jnp.int32(NPAD)),
                    axis=1, keepdims=True)
        cols.append(j)
        key = jnp.where(col == j, jnp.float32(jnp.inf), key)
    idx_ref[...] = jnp.concatenate(cols, axis=1)


def _knn_call(xbf, xbft, sqrow):
    return pl.pallas_call(
        _knn_body,
        grid=(NBLK,),
        in_specs=[
            pl.BlockSpec((BQ, D), lambda i: (i, 0)),
            pl.BlockSpec((D, NPAD), lambda i: (0, 0)),
            pl.BlockSpec((1, NPAD), lambda i: (0, 0)),
        ],
        out_specs=pl.BlockSpec((BQ, K), lambda i: (i, 0)),
        out_shape=jax.ShapeDtypeStruct((NPAD, K), jnp.int32),
    )(xbf, xbft, sqrow)


def _sc_body(a_hbm, b_hbm, idx_hbm, out_hbm, idx_v, rows_v, a_v, acc_v, sem):
    wid = lax.axis_index("s") * 2 + lax.axis_index("c")
    qbase = wid * QPW

    def batch_body(t, carry):
        q0 = qbase + t * QB
        pltpu.sync_copy(idx_hbm.at[pl.ds(q0 * K, QB * K)], idx_v)
        pltpu.async_copy(b_hbm.at[idx_v], rows_v, sem).wait()
        pltpu.sync_copy(a_hbm.at[pl.ds(q0, QB), :], a_v)
        for q in range(QB):
            for c in range(OUT // 16):
                av = a_v[q, pl.ds(c * 16, 16)]
                acc = jnp.zeros((16,), jnp.float32)
                for r in range(K):
                    bv = rows_v[q * K + r, pl.ds(c * 16, 16)]
                    acc = acc + jnp.maximum(av + bv, jnp.float32(0.0))
                acc_v[q, pl.ds(c * 16, 16)] = acc * jnp.float32(1.0 / K)
        pltpu.sync_copy(acc_v, out_hbm.at[pl.ds(q0, QB), :])
        return carry

    lax.fori_loop(0, NBATCH, batch_body, 0)


def _sc_call(A, B, idx_flat):
    f = functools.partial(
        pl.kernel,
        mesh=plsc.VectorSubcoreMesh(core_axis_name="c", subcore_axis_name="s"),
        out_type=jax.ShapeDtypeStruct((NPAD, OUT), jnp.float32),
        scratch_types=[
            pltpu.VMEM((QB * K,), jnp.int32),
            pltpu.VMEM((QB * K, OUT), jnp.float32),
            pltpu.VMEM((QB, OUT), jnp.float32),
            pltpu.VMEM((QB, OUT), jnp.float32),
            pltpu.SemaphoreType.DMA,
        ],
    )(_sc_body)
    return f(A, B, idx_flat)


def _w2_body(m_ref, w2_ref, b2_ref, out_ref):
    out_ref[...] = _mm3(m_ref[...], w2_ref[...]) + b2_ref[...]


def _w2_call(m, w2, b2row):
    return pl.pallas_call(
        _w2_body,
        out_shape=jax.ShapeDtypeStruct((NPAD, OUT), jnp.float32),
    )(m, w2, b2row)


def kernel(x, W1, b1, W2, b2):
    xp = jnp.pad(x, ((0, NPAD - N), (0, 0)))
    A, B, sqcol = _prep_call(xp, W1[:D], W1[D:], b1.reshape(1, OUT))
    xbf = xp.astype(jnp.bfloat16)
    idx = _knn_call(xbf, xbf.T, sqcol.reshape(1, NPAD))
    m = _sc_call(A, B, idx.reshape(NPAD * K))
    out = _w2_call(m, W2, b2.reshape(1, OUT))
    return out[:N]


# speedup vs baseline: 6.7049x; 1.4103x over previous
import functools

import jax
import jax.numpy as jnp
from jax import lax
from jax.experimental import pallas as pl
from jax.experimental.pallas import tpu as pltpu
from jax.experimental.pallas import tpu_sc as plsc

N = 10000
D = 128
K = 16
OUT = 128
NPAD = 10240
BQ = 256
NBLK = NPAD // BQ
NW = 32
QPW = NPAD // NW
QB = 8
NBATCH = QPW // QB


def _split_bf16(a):
    hi = a.astype(jnp.bfloat16)
    lo = (a - hi.astype(jnp.float32)).astype(jnp.bfloat16)
    return hi, lo


def _mm3(a, b):
    ah, al = _split_bf16(a)
    bh, bl = _split_bf16(b)
    return (jnp.dot(ah, bh, preferred_element_type=jnp.float32)
            + jnp.dot(ah, bl, preferred_element_type=jnp.float32)
            + jnp.dot(al, bh, preferred_element_type=jnp.float32))


def _prep_body(xp_ref, w1a_ref, w1b_ref, b1_ref, a_ref, b_ref, sq_ref):
    xp = xp_ref[...]
    wd = w1a_ref[...] - w1b_ref[...]
    a_ref[...] = _mm3(xp, wd) + b1_ref[...]
    b_ref[...] = _mm3(xp, w1b_ref[...])
    sq = jnp.sum(xp * xp, axis=1, keepdims=True)
    row = lax.broadcasted_iota(jnp.int32, (NPAD, 1), 0)
    sq_ref[...] = jnp.where(row < N, sq, jnp.float32(1e30))


def _prep_call(xp, w1a, w1b, b1row):
    return pl.pallas_call(
        _prep_body,
        out_shape=(
            jax.ShapeDtypeStruct((NPAD, OUT), jnp.float32),
            jax.ShapeDtypeStruct((NPAD, OUT), jnp.float32),
            jax.ShapeDtypeStruct((NPAD, 1), jnp.float32),
        ),
    )(xp, w1a, w1b, b1row)


SDEPTH = NPAD // 128
T0 = 5
T1 = 8


def _knn_body(xbf_ref, xbft_ref, sq_ref, idx_ref):
    dot = jnp.dot(xbf_ref[...], xbft_ref[...],
                  preferred_element_type=jnp.float32)
    key = sq_ref[...] - 2.0 * dot
    k3 = key.reshape(BQ, SDEPTH, 128)
    iota_d = lax.broadcasted_iota(jnp.int32, (BQ, SDEPTH, 128), 1)
    lane = lax.broadcasted_iota(jnp.int32, (BQ, 1, 128), 2)
    inf = jnp.float32(jnp.inf)
    bigj = jnp.int32(2**30)

    def round_(k3c):
        m = jnp.min(k3c, axis=1, keepdims=True)
        eq = k3c == m
        d = jnp.min(jnp.where(eq, iota_d, SDEPTH), axis=1, keepdims=True)
        return m, d, jnp.where(eq, inf, k3c)

    ms, ds = [], []
    k3c = k3
    for _ in range(T0):
        m, d, k3c = round_(k3c)
        ms.append(m)
        ds.append(d)

    def extract16(ms_, ds_):
        t = len(ms_)
        vf = jnp.concatenate(ms_, axis=1).reshape(BQ, t * 128)
        jf = jnp.concatenate([d * 128 + lane for d in ds_],
                             axis=1).reshape(BQ, t * 128)
        cols = []
        m = None
        for _ in range(K):
            m = jnp.min(vf, axis=1, keepdims=True)
            j = jnp.min(jnp.where(vf == m, jf, bigj), axis=1, keepdims=True)
            cols.append(j)
            vf = jnp.where(jf == j, inf, vf)
        return jnp.concatenate(cols, axis=1), m

    idx5, th = extract16(ms, ds)
    blockflag = jnp.any(ms[-1][:, 0, :] <= th)

    def fallback():
        msf, dsf = list(ms), list(ds)
        k3f = k3c
        for _ in range(T1 - T0):
            m, d, k3f = round_(k3f)
            msf.append(m)
            dsf.append(d)
        return extract16(msf, dsf)[0]

    idx_ref[...] = lax.cond(blockflag, fallback, lambda: idx5)


def _knn_call(xbf, xbft, sqrow):
    return pl.pallas_call(
        _knn_body,
        grid=(NBLK,),
        in_specs=[
            pl.BlockSpec((BQ, D), lambda i: (i, 0)),
            pl.BlockSpec((D, NPAD), lambda i: (0, 0)),
            pl.BlockSpec((1, NPAD), lambda i: (0, 0)),
        ],
        out_specs=pl.BlockSpec((BQ, K), lambda i: (i, 0)),
        out_shape=jax.ShapeDtypeStruct((NPAD, K), jnp.int32),
    )(xbf, xbft, sqrow)


def _sc_body(a_hbm, b_hbm, idx_hbm, out_hbm, idx_v, rows_v, a_v, acc_v, sem):
    wid = lax.axis_index("s") * 2 + lax.axis_index("c")
    qbase = wid * QPW

    def batch_body(t, carry):
        q0 = qbase + t * QB
        pltpu.sync_copy(idx_hbm.at[pl.ds(q0 * K, QB * K)], idx_v)
        pltpu.async_copy(b_hbm.at[idx_v], rows_v, sem).wait()
        pltpu.sync_copy(a_hbm.at[pl.ds(q0, QB), :], a_v)
        for q in range(QB):
            for c in range(OUT // 16):
                av = a_v[q, pl.ds(c * 16, 16)]
                acc = jnp.zeros((16,), jnp.float32)
                for r in range(K):
                    bv = rows_v[q * K + r, pl.ds(c * 16, 16)]
                    acc = acc + jnp.maximum(av + bv, jnp.float32(0.0))
                acc_v[q, pl.ds(c * 16, 16)] = acc * jnp.float32(1.0 / K)
        pltpu.sync_copy(acc_v, out_hbm.at[pl.ds(q0, QB), :])
        return carry

    lax.fori_loop(0, NBATCH, batch_body, 0)


def _sc_call(A, B, idx_flat):
    f = functools.partial(
        pl.kernel,
        mesh=plsc.VectorSubcoreMesh(core_axis_name="c", subcore_axis_name="s"),
        out_type=jax.ShapeDtypeStruct((NPAD, OUT), jnp.float32),
        scratch_types=[
            pltpu.VMEM((QB * K,), jnp.int32),
            pltpu.VMEM((QB * K, OUT), jnp.float32),
            pltpu.VMEM((QB, OUT), jnp.float32),
            pltpu.VMEM((QB, OUT), jnp.float32),
            pltpu.SemaphoreType.DMA,
        ],
    )(_sc_body)
    return f(A, B, idx_flat)


def _w2_body(m_ref, w2_ref, b2_ref, out_ref):
    out_ref[...] = _mm3(m_ref[...], w2_ref[...]) + b2_ref[...]


def _w2_call(m, w2, b2row):
    return pl.pallas_call(
        _w2_body,
        out_shape=jax.ShapeDtypeStruct((NPAD, OUT), jnp.float32),
    )(m, w2, b2row)


def kernel(x, W1, b1, W2, b2):
    xp = jnp.pad(x, ((0, NPAD - N), (0, 0)))
    A, B, sqcol = _prep_call(xp, W1[:D], W1[D:], b1.reshape(1, OUT))
    xbf = xp.astype(jnp.bfloat16)
    idx = _knn_call(xbf, xbf.T, sqcol.reshape(1, NPAD))
    m = _sc_call(A, B, idx.reshape(NPAD * K))
    out = _w2_call(m, W2, b2.reshape(1, OUT))
    return out[:N]
